# Initial kernel scaffold; baseline (speedup 1.0000x reference)
import jax
import jax.numpy as jnp
from jax.experimental import pallas as pl


def kernel(positions, node_attrs, W_embed, W_e0, W_radial, W_mix, W_sc, W_readout, scale, shift, edge_index, shifts, batch):
    raise NotImplementedError("write your pallas kernel here")


# speedup vs baseline: 1.8969x; 1.8969x over previous
import functools

import numpy as np
import jax
import jax.numpy as jnp
from jax import lax
from jax.experimental import pallas as pl
from jax.experimental.pallas import tpu as pltpu
from jax.experimental.pallas import tpu_sc as plsc

F32 = jnp.float32
N_NODES = 10000
N_EDGES = 320000
HIDDEN = 128
N_BESSEL = 8
N_GRAPHS = 16
R_MAX = 5.0
PADW = 16

NC, NS = 2, 16
NW = NC * NS
EPT = N_EDGES // NW
CHUNK = 80
NCHUNK = EPT // CHUNK
RPT = N_NODES // NS
RCH = 125

_HI = jax.lax.Precision.HIGHEST
_C0 = float(np.sqrt(2.0 / R_MAX))


def _bessel_freqs():
    n = lax.broadcasted_iota(jnp.int32, (1, N_BESSEL), 1).astype(F32) + 1.0
    return n * float(np.pi / R_MAX)


def _dot(a, b):
    return jnp.dot(a, b, precision=_HI, preferred_element_type=F32)


def _dott(a, b):
    return lax.dot_general(a, b, (((1,), (1,)), ((), ())),
                           precision=_HI, preferred_element_type=F32)


def _lazy(builder):
    cache = []

    def call(*args):
        if not cache:
            cache.append(builder())
        return cache[0](*args)

    return call


def _build_gather(D, n_idx):
    out_type = [jax.ShapeDtypeStruct((N_EDGES, D), F32) for _ in range(n_idx)]
    scratch_types = []
    for _ in range(n_idx):
        scratch_types += [pltpu.VMEM((CHUNK,), jnp.int32),
                          pltpu.VMEM((CHUNK, D), F32),
                          pltpu.SemaphoreType.DMA]

    mesh = plsc.VectorSubcoreMesh(core_axis_name="c", subcore_axis_name="s")

    @functools.partial(pl.kernel, mesh=mesh, out_type=out_type,
                       scratch_types=scratch_types,
                       compiler_params=pltpu.CompilerParams(
                           use_tc_tiling_on_sc=False))
    def gather_kernel(*refs):
        table = refs[0]
        idx_hbm = refs[1:1 + n_idx]
        outs = refs[1 + n_idx:1 + 2 * n_idx]
        scr = refs[1 + 2 * n_idx:]
        wid = lax.axis_index("c") * NS + lax.axis_index("s")
        base = wid * EPT

        def body(j, _):
            off = pl.multiple_of(base + j * CHUNK, 8)
            for k in range(n_idx):
                idx_v = scr[3 * k]
                rows_v = scr[3 * k + 1]
                sem = scr[3 * k + 2]
                pltpu.sync_copy(idx_hbm[k].at[pl.ds(off, CHUNK)], idx_v)
                pltpu.async_copy(table.at[idx_v], rows_v, sem).wait()
                pltpu.sync_copy(rows_v, outs[k].at[pl.ds(off, CHUNK)])
            return ()

        lax.fori_loop(0, NCHUNK, body, ())

    return gather_kernel


def _build_scatter(D, n_idx):
    out_type = [jax.ShapeDtypeStruct((NC, N_NODES, D), F32)
                for _ in range(n_idx)]
    scratch_types = [pltpu.VMEM((CHUNK, D), F32),
                     pltpu.VMEM((RCH, D), F32)]
    for _ in range(n_idx):
        scratch_types += [pltpu.VMEM((CHUNK,), jnp.int32),
                          pltpu.VMEM_SHARED((N_NODES, D), F32)]

    mesh = plsc.VectorSubcoreMesh(core_axis_name="c", subcore_axis_name="s")

    @functools.partial(pl.kernel, mesh=mesh, out_type=out_type,
                       scratch_types=scratch_types,
                       compiler_params=pltpu.CompilerParams(
                           use_tc_tiling_on_sc=False))
    def scatter_kernel(*refs):
        rows_hbm = refs[0]
        zeros_hbm = refs[1]
        idx_hbm = refs[2:2 + n_idx]
        outs = refs[2 + n_idx:2 + 2 * n_idx]
        rows_v = refs[2 + 2 * n_idx]
        stage_v = refs[3 + 2 * n_idx]
        idx_vs = [refs[4 + 2 * n_idx + 2 * k] for k in range(n_idx)]
        accs = [refs[5 + 2 * n_idx + 2 * k] for k in range(n_idx)]
        c = lax.axis_index("c")
        s = lax.axis_index("s")
        base = (c * NS + s) * EPT

        pltpu.sync_copy(zeros_hbm, stage_v)
        for i in range(RPT // RCH):
            r0 = s * RPT + i * RCH
            for k in range(n_idx):
                pltpu.sync_copy(stage_v, accs[k].at[pl.ds(r0, RCH)])
        plsc.subcore_barrier()

        def body(j, _):
            off = pl.multiple_of(base + j * CHUNK, 8)
            pltpu.sync_copy(rows_hbm.at[pl.ds(off, CHUNK)], rows_v)
            for k in range(n_idx):
                pltpu.sync_copy(idx_hbm[k].at[pl.ds(off, CHUNK)], idx_vs[k])
                pltpu.sync_copy(rows_v, accs[k].at[idx_vs[k]], add=True)
            return ()

        lax.fori_loop(0, NCHUNK, body, ())
        plsc.subcore_barrier()

        for i in range(RPT // RCH):
            r0 = s * RPT + i * RCH
            for k in range(n_idx):
                pltpu.sync_copy(accs[k].at[pl.ds(r0, RCH)], stage_v)
                pltpu.sync_copy(stage_v, outs[k].at[c, pl.ds(r0, RCH)])

    return scatter_kernel


_gather_pos = _lazy(lambda: _build_gather(PADW, 2))
_gather_feat = _lazy(lambda: _build_gather(HIDDEN, 1))
_scatter_q = _lazy(lambda: _build_scatter(HIDDEN, 1))
_scatter_f = _lazy(lambda: _build_scatter(PADW, 2))


_BN = 1000
_BE = 2000


def _node_prep(attrs_pad, we_pad, wsc_pad, we0_pad):
    def body(a_ref, we_ref, ws_ref, we0_ref, nf_ref, sc_ref, e0_ref):
        a = a_ref[...]
        nf_ref[...] = _dot(a, we_ref[...])
        sc_ref[...] = _dot(a, ws_ref[...])
        e0_ref[...] = _dot(a, we0_ref[...])

    return pl.pallas_call(
        body,
        grid=(N_NODES // _BN,),
        in_specs=[
            pl.BlockSpec((_BN, PADW), lambda i: (i, 0)),
            pl.BlockSpec((PADW, HIDDEN), lambda i: (0, 0)),
            pl.BlockSpec((PADW, HIDDEN), lambda i: (0, 0)),
            pl.BlockSpec((PADW, 1), lambda i: (0, 0)),
        ],
        out_specs=[
            pl.BlockSpec((_BN, HIDDEN), lambda i: (i, 0)),
            pl.BlockSpec((_BN, HIDDEN), lambda i: (i, 0)),
            pl.BlockSpec((_BN, 1), lambda i: (i, 0)),
        ],
        out_shape=[
            jax.ShapeDtypeStruct((N_NODES, HIDDEN), F32),
            jax.ShapeDtypeStruct((N_NODES, HIDDEN), F32),
            jax.ShapeDtypeStruct((N_NODES, 1), F32),
        ],
    )(attrs_pad, we_pad, wsc_pad, we0_pad)


def _edge_geometry(ps, pd, sf):
    v = pd - ps + sf
    l2 = jnp.sum(v * v, axis=1, keepdims=True) + 1e-12
    l = jnp.sqrt(l2)
    inv_l = 1.0 / l
    u = v * inv_l
    x = jnp.clip(l * (1.0 / R_MAX), 0.0, 1.0)
    cut = 1.0 - 10.0 * x ** 3 + 15.0 * x ** 4 - 6.0 * x ** 5
    freqs = _bessel_freqs()
    ang = l * freqs
    bes = _C0 * jnp.sin(ang) * inv_l
    ef = bes * cut
    return v, l, inv_l, u, x, cut, freqs, ang, bes, ef


def _edge_fwd(pos_src, pos_dst, shifts_pad, nf_src, wr, wm):
    def body(ps_ref, pd_ref, sf_ref, nf_ref, wr_ref, wm_ref, q_ref):
        _, _, _, u, _, _, _, _, _, ef = _edge_geometry(
            ps_ref[...], pd_ref[...], sf_ref[...])
        tp = _dot(ef, wr_ref[...])
        msg = nf_ref[...] * tp
        wmx = wm_ref[...]
        q = _dot(msg, wmx[0:HIDDEN])
        for i in range(3):
            q = q + u[:, i:i + 1] * _dot(msg, wmx[(i + 1) * HIDDEN:(i + 2) * HIDDEN])
        q_ref[...] = q

    return pl.pallas_call(
        body,
        grid=(N_EDGES // _BE,),
        in_specs=[
            pl.BlockSpec((_BE, PADW), lambda i: (i, 0)),
            pl.BlockSpec((_BE, PADW), lambda i: (i, 0)),
            pl.BlockSpec((_BE, PADW), lambda i: (i, 0)),
            pl.BlockSpec((_BE, HIDDEN), lambda i: (i, 0)),
            pl.BlockSpec((N_BESSEL, HIDDEN), lambda i: (0, 0)),
            pl.BlockSpec((4 * HIDDEN, HIDDEN), lambda i: (0, 0)),
        ],
        out_specs=pl.BlockSpec((_BE, HIDDEN), lambda i: (i, 0)),
        out_shape=jax.ShapeDtypeStruct((N_EDGES, HIDDEN), F32),
    )(pos_src, pos_dst, shifts_pad, nf_src, wr, wm)


def _node_out(pre_parts, sc_res, node_e0, wro, wro_row, scale2, shift2, batch_row):
    def body(pp_ref, sc_ref, e0_ref, wro_ref, wrr_ref, scl_ref, sft_ref,
             b_ref, nfo_ref, ne_ref, gp_ref, e0g_ref, ieg_ref):
        pre = jnp.sum(pp_ref[...], axis=0)
        t = jnp.tanh(pre)
        nfo = t + sc_ref[...]
        nfo_ref[...] = nfo
        es = _dot(nfo, wro_ref[...]) * scl_ref[...] + sft_ref[...]
        ne_ref[...] = e0_ref[...] + es
        gp_ref[...] = (1.0 - t * t) * (scl_ref[...] * wrr_ref[...])
        gids = lax.broadcasted_iota(jnp.int32, (N_GRAPHS, 1), 0)
        onehot_t = (gids == b_ref[...][0]).astype(F32)
        e0p = _dot(onehot_t, e0_ref[...])
        iep = _dot(onehot_t, es)
        first = pl.program_id(0) == 0

        @pl.when(first)
        def _():
            e0g_ref[...] = e0p
            ieg_ref[...] = iep

        @pl.when(jnp.logical_not(first))
        def _():
            e0g_ref[...] += e0p
            ieg_ref[...] += iep

    return pl.pallas_call(
        body,
        grid=(N_NODES // _BN,),
        in_specs=[
            pl.BlockSpec((NC, _BN, HIDDEN), lambda i: (0, i, 0)),
            pl.BlockSpec((_BN, HIDDEN), lambda i: (i, 0)),
            pl.BlockSpec((_BN, 1), lambda i: (i, 0)),
            pl.BlockSpec((HIDDEN, 1), lambda i: (0, 0)),
            pl.BlockSpec((1, HIDDEN), lambda i: (0, 0)),
            pl.BlockSpec((1, 1), lambda i: (0, 0)),
            pl.BlockSpec((1, 1), lambda i: (0, 0)),
            pl.BlockSpec((1, 1, _BN), lambda i: (i, 0, 0)),
        ],
        out_specs=[
            pl.BlockSpec((_BN, HIDDEN), lambda i: (i, 0)),
            pl.BlockSpec((_BN, 1), lambda i: (i, 0)),
            pl.BlockSpec((_BN, HIDDEN), lambda i: (i, 0)),
            pl.BlockSpec((N_GRAPHS, 1), lambda i: (0, 0)),
            pl.BlockSpec((N_GRAPHS, 1), lambda i: (0, 0)),
        ],
        out_shape=[
            jax.ShapeDtypeStruct((N_NODES, HIDDEN), F32),
            jax.ShapeDtypeStruct((N_NODES, 1), F32),
            jax.ShapeDtypeStruct((N_NODES, HIDDEN), F32),
            jax.ShapeDtypeStruct((N_GRAPHS, 1), F32),
            jax.ShapeDtypeStruct((N_GRAPHS, 1), F32),
        ],
    )(pre_parts, sc_res, node_e0, wro, wro_row, scale2, shift2, batch_row)


def _edge_bwd(pos_src, pos_dst, shifts_pad, nf_src, h_edge, wr, wm):
    def body(ps_ref, pd_ref, sf_ref, nf_ref, h_ref, wr_ref, wm_ref, gv_ref):
        v, l, inv_l, u, x, cut, freqs, ang, bes, ef = _edge_geometry(
            ps_ref[...], pd_ref[...], sf_ref[...])
        wrx = wr_ref[...]
        wmx = wm_ref[...]
        nf = nf_ref[...]
        h = h_ref[...]
        tp = _dot(ef, wrx)
        msg = nf * tp
        gmsg = _dott(h, wmx[0:HIDDEN])
        gsh = []
        for i in range(3):
            ai = _dott(h, wmx[(i + 1) * HIDDEN:(i + 2) * HIDDEN])
            gmsg = gmsg + u[:, i:i + 1] * ai
            gsh.append(jnp.sum(ai * msg, axis=1, keepdims=True))
        gtp = gmsg * nf
        gef = _dott(gtp, wrx)
        gb = gef * cut
        gcut = jnp.sum(gef * bes, axis=1, keepdims=True)
        db_dl = _C0 * freqs * jnp.cos(ang) * inv_l - bes * inv_l
        x3 = x * x * x
        dcut_dl = jnp.where(l < R_MAX,
                            (-30.0 * x * x + 60.0 * x3 - 30.0 * x3 * x) * (1.0 / R_MAX),
                            0.0)
        gl = jnp.sum(gb * db_dl, axis=1, keepdims=True) + gcut * dcut_dl
        gu16 = jnp.concatenate(
            gsh + [jnp.zeros((gsh[0].shape[0], PADW - 3), F32)], axis=1)
        dot_gu_v = (gsh[0] * v[:, 0:1] + gsh[1] * v[:, 1:2]
                    + gsh[2] * v[:, 2:3])
        gv_ref[...] = gu16 * inv_l + (gl - dot_gu_v * inv_l * inv_l) * v * inv_l

    return pl.pallas_call(
        body,
        grid=(N_EDGES // _BE,),
        in_specs=[
            pl.BlockSpec((_BE, PADW), lambda i: (i, 0)),
            pl.BlockSpec((_BE, PADW), lambda i: (i, 0)),
            pl.BlockSpec((_BE, PADW), lambda i: (i, 0)),
            pl.BlockSpec((_BE, HIDDEN), lambda i: (i, 0)),
            pl.BlockSpec((_BE, HIDDEN), lambda i: (i, 0)),
            pl.BlockSpec((N_BESSEL, HIDDEN), lambda i: (0, 0)),
            pl.BlockSpec((4 * HIDDEN, HIDDEN), lambda i: (0, 0)),
        ],
        out_specs=pl.BlockSpec((_BE, PADW), lambda i: (i, 0)),
        out_shape=jax.ShapeDtypeStruct((N_EDGES, PADW), F32),
    )(pos_src, pos_dst, shifts_pad, nf_src, h_edge, wr, wm)


def _forces_combine(acc_src, acc_dst):
    def body(s_ref, d_ref, f_ref):
        sarr = s_ref[...]
        darr = d_ref[...]
        f = sarr[0] + sarr[1] - darr[0] - darr[1]
        f_ref[...] = f[:, 0:3]

    return pl.pallas_call(
        body,
        grid=(N_NODES // _BN,),
        in_specs=[
            pl.BlockSpec((NC, _BN, PADW), lambda i: (0, i, 0)),
            pl.BlockSpec((NC, _BN, PADW), lambda i: (0, i, 0)),
        ],
        out_specs=pl.BlockSpec((_BN, 3), lambda i: (i, 0)),
        out_shape=jax.ShapeDtypeStruct((N_NODES, 3), F32),
    )(acc_src, acc_dst)


def kernel(positions, node_attrs, W_embed, W_e0, W_radial, W_mix, W_sc,
           W_readout, scale, shift, edge_index, shifts, batch):
    src = edge_index[0].astype(jnp.int32)
    dst = edge_index[1].astype(jnp.int32)
    pos_pad = jnp.pad(positions, ((0, 0), (0, PADW - 3)))
    shifts_pad = jnp.pad(shifts, ((0, 0), (0, PADW - 3)))
    attrs_pad = jnp.pad(node_attrs, ((0, 0), (0, PADW - node_attrs.shape[1])))
    we_pad = jnp.pad(W_embed, ((0, PADW - W_embed.shape[0]), (0, 0)))
    wsc_pad = jnp.pad(W_sc, ((0, PADW - W_sc.shape[0]), (0, 0)))
    we0_pad = jnp.pad(W_e0.reshape(-1, 1), ((0, PADW - W_e0.shape[0]), (0, 0)))
    wro = W_readout.reshape(HIDDEN, 1)
    wro_row = W_readout.reshape(1, HIDDEN)
    scale2 = scale.reshape(1, 1).astype(F32)
    shift2 = shift.reshape(1, 1).astype(F32)
    batch_row = batch.reshape(N_NODES // _BN, 1, _BN).astype(jnp.int32)
    zeros_h = jnp.zeros((RCH, HIDDEN), F32)
    zeros_w = jnp.zeros((RCH, PADW), F32)

    node_feats, sc_res, node_e0 = _node_prep(attrs_pad, we_pad, wsc_pad, we0_pad)
    pos_src, pos_dst = _gather_pos(pos_pad, src, dst)
    (nf_src,) = _gather_feat(node_feats, src)
    q = _edge_fwd(pos_src, pos_dst, shifts_pad, nf_src, W_radial, W_mix)
    (pre_parts,) = _scatter_q(q, zeros_h, dst)
    nfo, node_energy, gpre, e0g, ieg = _node_out(
        pre_parts, sc_res, node_e0, wro, wro_row, scale2, shift2, batch_row)
    (h_edge,) = _gather_feat(gpre, dst)
    gvec = _edge_bwd(pos_src, pos_dst, shifts_pad, nf_src, h_edge,
                     W_radial, W_mix)
    acc_src, acc_dst = _scatter_f(gvec, zeros_w, src, dst)
    forces = _forces_combine(acc_src, acc_dst)

    total_energy = (e0g + ieg).reshape(N_GRAPHS)
    inter_e = ieg.reshape(N_GRAPHS)
    node_energy = node_energy.reshape(N_NODES)
    return total_energy, node_energy, inter_e, forces, nfo
